# Initial kernel scaffold; baseline (speedup 1.0000x reference)
import jax
import jax.numpy as jnp
from jax.experimental import pallas as pl


def kernel(node_features, edge_index, Ws0, bs0, Wc0, bc0, A1_0, b1_0, A2_0, g0, be0, rm0, rv0, Ws1, bs1, Wc1, bc1, A1_1, b1_1, A2_1, g1, be1, rm1, rv1, D1, d1b, D2, d2b):
    raise NotImplementedError("write your pallas kernel here")


# speedup vs baseline: 1.6447x; 1.6447x over previous
import functools

import jax
import jax.numpy as jnp
from jax import lax
from jax.experimental import pallas as pl

N = 10000
E = 160000
D = 128
H = 128
AH = 16

ROWS = 1000


def _dot_t(a, b):
    return lax.dot_general(a, b, (((1,), (1,)), ((), ())),
                           preferred_element_type=jnp.float32)


def _attention(self_v, outgoing, incoming, A1, b1, A2):
    A2r = A2.astype(jnp.bfloat16).astype(jnp.float32)
    ws = []
    for v in (self_v, outgoing, incoming):
        h = jnp.tanh(_dot_t(v, A1) + b1)
        hr = h.astype(jnp.bfloat16).astype(jnp.float32)
        s = _dot_t(hr, A2r)
        ws.append(jnp.tanh(s))
    w0, w1, w2 = ws
    fused = w0 * self_v + w1 * outgoing + w2 * incoming
    return fused, w0, w1, w2


def _bn_relu(x, g, be, rm, rv):
    return jnp.maximum((x - rm) / jnp.sqrt(rv + 1e-5) * g + be, 0.0)


def _layer0_body(x_ref, gin_ref, gout_ref, din_ref, dout_ref,
                 Ws_ref, bs_ref, Wc_ref, bc_ref, A1_ref, b1_ref, A2_ref,
                 g_ref, be_ref, rm_ref, rv_ref,
                 x1_ref, att_ref):
    x = x_ref[...]
    din = din_ref[:, 0:1]
    dout = dout_ref[:, 0:1]
    gin = gin_ref[...]
    gout = gout_ref[...]
    Wc = Wc_ref[...]
    WcA = Wc[:, :D]
    WcB = Wc[:, D:]
    bc = bc_ref[...]
    self_v = _dot_t(x, Ws_ref[...]) + bs_ref[...]
    outgoing = _dot_t(din * x - gin, WcA) + _dot_t(gin, WcB) + bc
    incoming = _dot_t(dout * x - gout, WcA) + _dot_t(gout, WcB) + bc
    fused, w0, w1, w2 = _attention(self_v, outgoing, incoming,
                                   A1_ref[...], b1_ref[...], A2_ref[...])
    x1_ref[...] = _bn_relu(fused, g_ref[...], be_ref[...], rm_ref[...], rv_ref[...])
    att_ref[...] = jnp.concatenate([w0, w1, w2], axis=1)


def _layer1_body(x_ref, gin_ref, gout_ref, din_ref, dout_ref,
                 Ws_ref, bs_ref, Wc_ref, bc_ref, A1_ref, b1_ref, A2_ref,
                 g_ref, be_ref, rm_ref, rv_ref,
                 D1_ref, d1b_ref, D2_ref, d2b_ref,
                 logp_ref):
    x = x_ref[...]
    din = din_ref[:, 0:1]
    dout = dout_ref[:, 0:1]
    gin = gin_ref[...]
    gout = gout_ref[...]
    Wc = Wc_ref[...]
    WcA = Wc[:, :H]
    WcB = Wc[:, H:]
    bc = bc_ref[...]
    self_v = _dot_t(x, Ws_ref[...]) + bs_ref[...]
    outgoing = _dot_t(din * x - gin, WcA) + _dot_t(gin, WcB) + bc
    incoming = _dot_t(dout * x - gout, WcA) + _dot_t(gout, WcB) + bc
    fused, _, _, _ = _attention(self_v, outgoing, incoming,
                                A1_ref[...], b1_ref[...], A2_ref[...])
    y = _bn_relu(fused, g_ref[...], be_ref[...], rm_ref[...], rv_ref[...])
    z = jnp.maximum(_dot_t(y, D1_ref[...]) + d1b_ref[...], 0.0)
    logits = _dot_t(z, D2_ref[...]) + d2b_ref[...]
    shifted = logits - jnp.max(logits, axis=1, keepdims=True)
    logp_ref[...] = shifted - jnp.log(
        jnp.sum(jnp.exp(shifted), axis=1, keepdims=True))


def _row_spec(cols):
    return pl.BlockSpec((ROWS, cols), lambda i: (i, 0))


def _full_spec(shape):
    return pl.BlockSpec(shape, lambda i: tuple(0 for _ in shape))


def _dense_layer0(x, gin, gout, din16, dout16, Ws, bs, Wc, bc, A1, b1, A2,
                  g, be, rm, rv):
    grid = (N // ROWS,)
    in_specs = [
        _row_spec(D), _row_spec(D), _row_spec(D), _row_spec(16), _row_spec(16),
        _full_spec((H, D)), _full_spec((1, H)), _full_spec((H, 2 * D)),
        _full_spec((1, H)), _full_spec((AH, H)), _full_spec((1, AH)),
        _full_spec((1, AH)),
        _full_spec((1, H)), _full_spec((1, H)), _full_spec((1, H)),
        _full_spec((1, H)),
    ]
    out_specs = [_row_spec(H), _row_spec(3)]
    out_shape = [jax.ShapeDtypeStruct((N, H), jnp.float32),
                 jax.ShapeDtypeStruct((N, 3), jnp.float32)]
    return pl.pallas_call(
        _layer0_body, grid=grid, in_specs=in_specs, out_specs=out_specs,
        out_shape=out_shape,
    )(x, gin, gout, din16, dout16, Ws, bs.reshape(1, -1), Wc,
      bc.reshape(1, -1), A1, b1.reshape(1, -1), A2,
      g.reshape(1, -1), be.reshape(1, -1), rm.reshape(1, -1), rv.reshape(1, -1))


def _dense_layer1(x, gin, gout, din16, dout16, Ws, bs, Wc, bc, A1, b1, A2,
                  g, be, rm, rv, D1, d1b, D2, d2b):
    grid = (N // ROWS,)
    in_specs = [
        _row_spec(H), _row_spec(H), _row_spec(H), _row_spec(16), _row_spec(16),
        _full_spec((H, H)), _full_spec((1, H)), _full_spec((H, 2 * H)),
        _full_spec((1, H)), _full_spec((AH, H)), _full_spec((1, AH)),
        _full_spec((1, AH)),
        _full_spec((1, H)), _full_spec((1, H)), _full_spec((1, H)),
        _full_spec((1, H)),
        _full_spec((H, H)), _full_spec((1, H)), _full_spec((2, H)),
        _full_spec((1, 2)),
    ]
    out_specs = [_row_spec(2)]
    out_shape = [jax.ShapeDtypeStruct((N, 2), jnp.float32)]
    return pl.pallas_call(
        _layer1_body, grid=grid, in_specs=in_specs, out_specs=out_specs,
        out_shape=out_shape,
    )(x, gin, gout, din16, dout16, Ws, bs.reshape(1, -1), Wc,
      bc.reshape(1, -1), A1, b1.reshape(1, -1), A2,
      g.reshape(1, -1), be.reshape(1, -1), rm.reshape(1, -1), rv.reshape(1, -1),
      D1, d1b.reshape(1, -1), D2, d2b.reshape(1, -1))[0]


def _scatter_sums(x, src, dst):
    gin = jax.ops.segment_sum(x[src], dst, num_segments=N)
    gout = jax.ops.segment_sum(x[dst], src, num_segments=N)
    return gin, gout


def kernel(node_features, edge_index, Ws0, bs0, Wc0, bc0, A1_0, b1_0, A2_0,
           g0, be0, rm0, rv0, Ws1, bs1, Wc1, bc1, A1_1, b1_1, A2_1, g1, be1,
           rm1, rv1, D1, d1b, D2, d2b):
    src = edge_index[0]
    dst = edge_index[1]
    x = node_features

    ones = jnp.ones((E,), jnp.float32)
    din = jax.ops.segment_sum(ones, dst, num_segments=N)
    dout = jax.ops.segment_sum(ones, src, num_segments=N)
    din16 = jnp.broadcast_to(din[:, None], (N, 16))
    dout16 = jnp.broadcast_to(dout[:, None], (N, 16))

    gin0, gout0 = _scatter_sums(x, src, dst)
    x1, att = _dense_layer0(x, gin0, gout0, din16, dout16, Ws0, bs0, Wc0,
                            bc0, A1_0, b1_0, A2_0, g0, be0, rm0, rv0)
    gin1, gout1 = _scatter_sums(x1, src, dst)
    logp = _dense_layer1(x1, gin1, gout1, din16, dout16, Ws1, bs1, Wc1, bc1,
                         A1_1, b1_1, A2_1, g1, be1, rm1, rv1, D1, d1b, D2, d2b)
    return (logp, att.reshape(N, 3, 1))


# speedup vs baseline: 1.9885x; 1.2090x over previous
import functools

import jax
import jax.numpy as jnp
from jax import lax
from jax.experimental import pallas as pl
from jax.experimental.pallas import tpu as pltpu
from jax.experimental.pallas import tpu_sc as plsc

N = 10000
E = 160000
D = 128
H = 128
AH = 16

ROWS = 1000

NS = 16
EP = E // NS
CH = 16
T = EP // CH
NP = 10240
RP = NP // NS
RZ = 40
DZ = 80


def _dot_t(a, b):
    return lax.dot_general(a, b, (((1,), (1,)), ((), ())),
                           preferred_element_type=jnp.float32)


def _attention(self_v, outgoing, incoming, A1, b1, A2):
    A2r = A2.astype(jnp.bfloat16).astype(jnp.float32)
    ws = []
    for v in (self_v, outgoing, incoming):
        h = jnp.tanh(_dot_t(v, A1) + b1)
        hr = h.astype(jnp.bfloat16).astype(jnp.float32)
        s = _dot_t(hr, A2r)
        ws.append(jnp.tanh(s))
    w0, w1, w2 = ws
    fused = w0 * self_v + w1 * outgoing + w2 * incoming
    return fused, w0, w1, w2


def _bn_relu(x, g, be, rm, rv):
    return jnp.maximum((x - rm) / jnp.sqrt(rv + 1e-5) * g + be, 0.0)


def _layer0_body(x_ref, gin_ref, gout_ref, din_ref, dout_ref,
                 Ws_ref, bs_ref, Wc_ref, bc_ref, A1_ref, b1_ref, A2_ref,
                 g_ref, be_ref, rm_ref, rv_ref,
                 x1_ref, att_ref):
    x = x_ref[...]
    din = din_ref[:, 0:1]
    dout = dout_ref[:, 0:1]
    gin = gin_ref[...]
    gout = gout_ref[...]
    Wc = Wc_ref[...]
    WcA = Wc[:, :D]
    WcB = Wc[:, D:]
    bc = bc_ref[...]
    self_v = _dot_t(x, Ws_ref[...]) + bs_ref[...]
    outgoing = _dot_t(din * x - gin, WcA) + _dot_t(gin, WcB) + bc
    incoming = _dot_t(dout * x - gout, WcA) + _dot_t(gout, WcB) + bc
    fused, w0, w1, w2 = _attention(self_v, outgoing, incoming,
                                   A1_ref[...], b1_ref[...], A2_ref[...])
    x1_ref[...] = _bn_relu(fused, g_ref[...], be_ref[...], rm_ref[...], rv_ref[...])
    att_ref[...] = jnp.concatenate([w0, w1, w2], axis=1)


def _layer1_body(x_ref, gin_ref, gout_ref, din_ref, dout_ref,
                 Ws_ref, bs_ref, Wc_ref, bc_ref, A1_ref, b1_ref, A2_ref,
                 g_ref, be_ref, rm_ref, rv_ref,
                 D1_ref, d1b_ref, D2_ref, d2b_ref,
                 logp_ref):
    x = x_ref[...]
    din = din_ref[:, 0:1]
    dout = dout_ref[:, 0:1]
    gin = gin_ref[...]
    gout = gout_ref[...]
    Wc = Wc_ref[...]
    WcA = Wc[:, :H]
    WcB = Wc[:, H:]
    bc = bc_ref[...]
    self_v = _dot_t(x, Ws_ref[...]) + bs_ref[...]
    outgoing = _dot_t(din * x - gin, WcA) + _dot_t(gin, WcB) + bc
    incoming = _dot_t(dout * x - gout, WcA) + _dot_t(gout, WcB) + bc
    fused, _, _, _ = _attention(self_v, outgoing, incoming,
                                A1_ref[...], b1_ref[...], A2_ref[...])
    y = _bn_relu(fused, g_ref[...], be_ref[...], rm_ref[...], rv_ref[...])
    z = jnp.maximum(_dot_t(y, D1_ref[...]) + d1b_ref[...], 0.0)
    logits = _dot_t(z, D2_ref[...]) + d2b_ref[...]
    shifted = logits - jnp.max(logits, axis=1, keepdims=True)
    logp_ref[...] = shifted - jnp.log(
        jnp.sum(jnp.exp(shifted), axis=1, keepdims=True))


def _row_spec(cols):
    return pl.BlockSpec((ROWS, cols), lambda i: (i, 0))


def _full_spec(shape):
    return pl.BlockSpec(shape, lambda i: tuple(0 for _ in shape))


def _dense_layer0(x, gin, gout, din16, dout16, Ws, bs, Wc, bc, A1, b1, A2,
                  g, be, rm, rv):
    grid = (N // ROWS,)
    in_specs = [
        _row_spec(D), _row_spec(D), _row_spec(D), _row_spec(16), _row_spec(16),
        _full_spec((H, D)), _full_spec((1, H)), _full_spec((H, 2 * D)),
        _full_spec((1, H)), _full_spec((AH, H)), _full_spec((1, AH)),
        _full_spec((1, AH)),
        _full_spec((1, H)), _full_spec((1, H)), _full_spec((1, H)),
        _full_spec((1, H)),
    ]
    out_specs = [_row_spec(H), _row_spec(3)]
    out_shape = [jax.ShapeDtypeStruct((N, H), jnp.float32),
                 jax.ShapeDtypeStruct((N, 3), jnp.float32)]
    return pl.pallas_call(
        _layer0_body, grid=grid, in_specs=in_specs, out_specs=out_specs,
        out_shape=out_shape,
    )(x, gin, gout, din16, dout16, Ws, bs.reshape(1, -1), Wc,
      bc.reshape(1, -1), A1, b1.reshape(1, -1), A2,
      g.reshape(1, -1), be.reshape(1, -1), rm.reshape(1, -1), rv.reshape(1, -1))


def _dense_layer1(x, gin, gout, din16, dout16, Ws, bs, Wc, bc, A1, b1, A2,
                  g, be, rm, rv, D1, d1b, D2, d2b):
    grid = (N // ROWS,)
    in_specs = [
        _row_spec(H), _row_spec(H), _row_spec(H), _row_spec(16), _row_spec(16),
        _full_spec((H, H)), _full_spec((1, H)), _full_spec((H, 2 * H)),
        _full_spec((1, H)), _full_spec((AH, H)), _full_spec((1, AH)),
        _full_spec((1, AH)),
        _full_spec((1, H)), _full_spec((1, H)), _full_spec((1, H)),
        _full_spec((1, H)),
        _full_spec((H, H)), _full_spec((1, H)), _full_spec((2, H)),
        _full_spec((1, 2)),
    ]
    out_specs = [_row_spec(2)]
    out_shape = [jax.ShapeDtypeStruct((N, 2), jnp.float32)]
    return pl.pallas_call(
        _layer1_body, grid=grid, in_specs=in_specs, out_specs=out_specs,
        out_shape=out_shape,
    )(x, gin, gout, din16, dout16, Ws, bs.reshape(1, -1), Wc,
      bc.reshape(1, -1), A1, b1.reshape(1, -1), A2,
      g.reshape(1, -1), be.reshape(1, -1), rm.reshape(1, -1), rv.reshape(1, -1),
      D1, d1b.reshape(1, -1), D2, d2b.reshape(1, -1))[0]


def _make_sc_scatter(with_deg):
    mesh = plsc.VectorSubcoreMesh(core_axis_name="c", subcore_axis_name="s")
    out_type = [jax.ShapeDtypeStruct((2 * NP, H), jnp.float32)]
    scratch = [
        pltpu.VMEM_SHARED((NP, H), jnp.float32),
        pltpu.VMEM((CH,), jnp.int32),
        pltpu.VMEM((CH,), jnp.int32),
        pltpu.VMEM((CH, H), jnp.float32),
        pltpu.VMEM((RZ, H), jnp.float32),
        pltpu.SemaphoreType.DMA,
    ]
    if with_deg:
        out_type.append(jax.ShapeDtypeStruct((2 * NP, 16), jnp.float32))
        scratch += [
            pltpu.VMEM_SHARED((NP, 16), jnp.float32),
            pltpu.VMEM((DZ, 16), jnp.float32),
            pltpu.VMEM((CH, 16), jnp.float32),
        ]

    def body(x_hbm, gidx_hbm, sidx_hbm, gsum_hbm, *rest):
        if with_deg:
            (deg_hbm, acc, gidx_v, sidx_v, rows_v, zbuf, sem,
             degacc, zdeg, ones_v) = rest
        else:
            acc, gidx_v, sidx_v, rows_v, zbuf, sem = rest
        cid = lax.axis_index("c")
        sid = lax.axis_index("s")
        row0 = sid * RP

        def zero_zbuf(k, carry):
            zbuf[k // 8, pl.ds((k % 8) * 16, 16)] = jnp.zeros((16,), jnp.float32)
            return carry
        lax.fori_loop(0, RZ * 8, zero_zbuf, 0)
        for r in range(RP // RZ):
            pltpu.sync_copy(zbuf, acc.at[pl.ds(row0 + r * RZ, RZ)])
        if with_deg:
            def zero_zdeg(k, carry):
                zdeg[k] = jnp.zeros((16,), jnp.float32)
                return carry
            lax.fori_loop(0, DZ, zero_zdeg, 0)
            for r in range(RP // DZ):
                pltpu.sync_copy(zdeg, degacc.at[pl.ds(row0 + r * DZ, DZ)])

            def fill_ones(k, carry):
                ones_v[k] = jnp.full((16,), 1.0, jnp.float32)
                return carry
            lax.fori_loop(0, CH, fill_ones, 0)
        plsc.subcore_barrier()

        base = cid * E + sid * EP

        def step(j, carry):
            off = base + j * CH
            pltpu.sync_copy(gidx_hbm.at[pl.ds(off, CH)], gidx_v)
            pltpu.sync_copy(sidx_hbm.at[pl.ds(off, CH)], sidx_v)
            pltpu.async_copy(x_hbm.at[gidx_v], rows_v, sem).wait()
            pltpu.sync_copy(rows_v, acc.at[sidx_v], add=True)
            if with_deg:
                pltpu.sync_copy(ones_v, degacc.at[sidx_v], add=True)
            return carry
        lax.fori_loop(0, T, step, 0)

        plsc.subcore_barrier()
        pltpu.sync_copy(acc.at[pl.ds(row0, RP)],
                        gsum_hbm.at[pl.ds(cid * NP + row0, RP)])
        if with_deg:
            pltpu.sync_copy(degacc.at[pl.ds(row0, RP)],
                            deg_hbm.at[pl.ds(cid * NP + row0, RP)])

    return pl.kernel(body, out_type=out_type, mesh=mesh,
                     scratch_types=scratch)


@functools.lru_cache(maxsize=None)
def _sc_scatter_fn(with_deg):
    return _make_sc_scatter(with_deg)


def kernel(node_features, edge_index, Ws0, bs0, Wc0, bc0, A1_0, b1_0, A2_0,
           g0, be0, rm0, rv0, Ws1, bs1, Wc1, bc1, A1_1, b1_1, A2_1, g1, be1,
           rm1, rv1, D1, d1b, D2, d2b):
    x = node_features
    src = edge_index[0]
    dst = edge_index[1]
    gidx = jnp.concatenate([src, dst])
    sidx = jnp.concatenate([dst, src])

    (g0s,) = _sc_scatter_fn(False)(x, gidx, sidx)
    g0s = g0s.reshape(2, NP, H)
    gin0, gout0 = g0s[0, :N], g0s[1, :N]
    ones_e = jnp.ones((E,), jnp.float32)
    din = jnp.zeros((N,), jnp.float32).at[dst].add(ones_e)
    dout = jnp.zeros((N,), jnp.float32).at[src].add(ones_e)
    din16 = jnp.broadcast_to(din[:, None], (N, 16))
    dout16 = jnp.broadcast_to(dout[:, None], (N, 16))
    x1, att = _dense_layer0(x, gin0, gout0, din16, dout16, Ws0, bs0, Wc0,
                            bc0, A1_0, b1_0, A2_0, g0, be0, rm0, rv0)
    (g1s,) = _sc_scatter_fn(False)(x1, gidx, sidx)
    g1s = g1s.reshape(2, NP, H)
    logp = _dense_layer1(x1, g1s[0, :N], g1s[1, :N], din16, dout16, Ws1, bs1,
                         Wc1, bc1, A1_1, b1_1, A2_1, g1, be1, rm1, rv1,
                         D1, d1b, D2, d2b)
    return (logp, att.reshape(N, 3, 1))


# speedup vs baseline: 3.0929x; 1.5554x over previous
import functools

import jax
import jax.numpy as jnp
from jax import lax
from jax.experimental import pallas as pl
from jax.experimental.pallas import tpu as pltpu
from jax.experimental.pallas import tpu_sc as plsc

N = 10000
E = 160000
D = 128
H = 128
AH = 16

ROWS = 1000

NS = 16
EP = E // NS
CH = 16
T = EP // CH
NP = 10240
RP = NP // NS
RZ = 40


def _dot_t(a, b):
    return lax.dot_general(a, b, (((1,), (1,)), ((), ())),
                           preferred_element_type=jnp.float32)


def _attention(self_v, outgoing, incoming, A1, b1, A2):
    A2r = A2.astype(jnp.bfloat16).astype(jnp.float32)
    ws = []
    for v in (self_v, outgoing, incoming):
        h = jnp.tanh(_dot_t(v, A1) + b1)
        hr = h.astype(jnp.bfloat16).astype(jnp.float32)
        s = _dot_t(hr, A2r)
        ws.append(jnp.tanh(s))
    w0, w1, w2 = ws
    fused = w0 * self_v + w1 * outgoing + w2 * incoming
    return fused, w0, w1, w2


def _bn_relu(x, g, be, rm, rv):
    return jnp.maximum((x - rm) / jnp.sqrt(rv + 1e-5) * g + be, 0.0)


def _layer0_body(x_ref, gin_ref, gout_ref, din_ref, dout_ref,
                 Ws_ref, bs_ref, Wc_ref, bc_ref, A1_ref, b1_ref, A2_ref,
                 g_ref, be_ref, rm_ref, rv_ref,
                 x1_ref, att_ref):
    x = x_ref[...]
    din = din_ref[:, 0:1]
    dout = dout_ref[:, 0:1]
    gin = gin_ref[...]
    gout = gout_ref[...]
    Wc = Wc_ref[...]
    WcA = Wc[:, :D]
    WcB = Wc[:, D:]
    bc = bc_ref[...]
    self_v = _dot_t(x, Ws_ref[...]) + bs_ref[...]
    outgoing = _dot_t(din * x - gin, WcA) + _dot_t(gin, WcB) + bc
    incoming = _dot_t(dout * x - gout, WcA) + _dot_t(gout, WcB) + bc
    fused, w0, w1, w2 = _attention(self_v, outgoing, incoming,
                                   A1_ref[...], b1_ref[...], A2_ref[...])
    x1_ref[...] = _bn_relu(fused, g_ref[...], be_ref[...], rm_ref[...], rv_ref[...])
    att_ref[...] = jnp.concatenate([w0, w1, w2], axis=1)


def _layer1_body(x_ref, gin_ref, gout_ref, din_ref, dout_ref,
                 Ws_ref, bs_ref, Wc_ref, bc_ref, A1_ref, b1_ref, A2_ref,
                 g_ref, be_ref, rm_ref, rv_ref,
                 D1_ref, d1b_ref, D2_ref, d2b_ref,
                 logp_ref):
    x = x_ref[...]
    din = din_ref[:, 0:1]
    dout = dout_ref[:, 0:1]
    gin = gin_ref[...]
    gout = gout_ref[...]
    Wc = Wc_ref[...]
    WcA = Wc[:, :H]
    WcB = Wc[:, H:]
    bc = bc_ref[...]
    self_v = _dot_t(x, Ws_ref[...]) + bs_ref[...]
    outgoing = _dot_t(din * x - gin, WcA) + _dot_t(gin, WcB) + bc
    incoming = _dot_t(dout * x - gout, WcA) + _dot_t(gout, WcB) + bc
    fused, _, _, _ = _attention(self_v, outgoing, incoming,
                                A1_ref[...], b1_ref[...], A2_ref[...])
    y = _bn_relu(fused, g_ref[...], be_ref[...], rm_ref[...], rv_ref[...])
    z = jnp.maximum(_dot_t(y, D1_ref[...]) + d1b_ref[...], 0.0)
    logits = _dot_t(z, D2_ref[...]) + d2b_ref[...]
    shifted = logits - jnp.max(logits, axis=1, keepdims=True)
    logp_ref[...] = shifted - jnp.log(
        jnp.sum(jnp.exp(shifted), axis=1, keepdims=True))


def _row_spec(cols):
    return pl.BlockSpec((ROWS, cols), lambda i: (i, 0))


def _full_spec(shape):
    return pl.BlockSpec(shape, lambda i: tuple(0 for _ in shape))


def _dense_layer0(x, gin, gout, din16, dout16, Ws, bs, Wc, bc, A1, b1, A2,
                  g, be, rm, rv):
    grid = (N // ROWS,)
    in_specs = [
        _row_spec(D), _row_spec(D), _row_spec(D), _row_spec(16), _row_spec(16),
        _full_spec((H, D)), _full_spec((1, H)), _full_spec((H, 2 * D)),
        _full_spec((1, H)), _full_spec((AH, H)), _full_spec((1, AH)),
        _full_spec((1, AH)),
        _full_spec((1, H)), _full_spec((1, H)), _full_spec((1, H)),
        _full_spec((1, H)),
    ]
    out_specs = [_row_spec(H), _row_spec(3)]
    out_shape = [jax.ShapeDtypeStruct((N, H), jnp.float32),
                 jax.ShapeDtypeStruct((N, 3), jnp.float32)]
    return pl.pallas_call(
        _layer0_body, grid=grid, in_specs=in_specs, out_specs=out_specs,
        out_shape=out_shape,
    )(x, gin, gout, din16, dout16, Ws, bs.reshape(1, -1), Wc,
      bc.reshape(1, -1), A1, b1.reshape(1, -1), A2,
      g.reshape(1, -1), be.reshape(1, -1), rm.reshape(1, -1), rv.reshape(1, -1))


def _dense_layer1(x, gin, gout, din16, dout16, Ws, bs, Wc, bc, A1, b1, A2,
                  g, be, rm, rv, D1, d1b, D2, d2b):
    grid = (N // ROWS,)
    in_specs = [
        _row_spec(H), _row_spec(H), _row_spec(H), _row_spec(16), _row_spec(16),
        _full_spec((H, H)), _full_spec((1, H)), _full_spec((H, 2 * H)),
        _full_spec((1, H)), _full_spec((AH, H)), _full_spec((1, AH)),
        _full_spec((1, AH)),
        _full_spec((1, H)), _full_spec((1, H)), _full_spec((1, H)),
        _full_spec((1, H)),
        _full_spec((H, H)), _full_spec((1, H)), _full_spec((2, H)),
        _full_spec((1, 2)),
    ]
    out_specs = [_row_spec(2)]
    out_shape = [jax.ShapeDtypeStruct((N, 2), jnp.float32)]
    return pl.pallas_call(
        _layer1_body, grid=grid, in_specs=in_specs, out_specs=out_specs,
        out_shape=out_shape,
    )(x, gin, gout, din16, dout16, Ws, bs.reshape(1, -1), Wc,
      bc.reshape(1, -1), A1, b1.reshape(1, -1), A2,
      g.reshape(1, -1), be.reshape(1, -1), rm.reshape(1, -1), rv.reshape(1, -1),
      D1, d1b.reshape(1, -1), D2, d2b.reshape(1, -1))[0]


def _make_sc_scatter():
    mesh = plsc.VectorSubcoreMesh(core_axis_name="c", subcore_axis_name="s")
    out_type = [jax.ShapeDtypeStruct((2 * NP, H), jnp.float32)]
    scratch = [
        pltpu.VMEM_SHARED((NP, H), jnp.float32),
        pltpu.VMEM((CH,), jnp.int32),
        pltpu.VMEM((CH,), jnp.int32),
        pltpu.VMEM((CH,), jnp.int32),
        pltpu.VMEM((CH,), jnp.int32),
        pltpu.VMEM((CH, H), jnp.float32),
        pltpu.VMEM((CH, H), jnp.float32),
        pltpu.VMEM((RZ, H), jnp.float32),
        pltpu.SemaphoreType.DMA,
        pltpu.SemaphoreType.DMA,
    ]

    def body(x_hbm, gidx_hbm, sidx_hbm, gsum_hbm,
             acc, g0, s0, g1, s1, r0, r1, zbuf, m0, m1):
        cid = lax.axis_index("c")
        sid = lax.axis_index("s")
        row0 = sid * RP

        def zero_zbuf(k, carry):
            zbuf[k // 8, pl.ds((k % 8) * 16, 16)] = jnp.zeros((16,), jnp.float32)
            return carry
        lax.fori_loop(0, RZ * 8, zero_zbuf, 0)
        for r in range(RP // RZ):
            pltpu.sync_copy(zbuf, acc.at[pl.ds(row0 + r * RZ, RZ)])
        plsc.subcore_barrier()

        base = cid * E + sid * EP

        def fetch(j, g, s, r, m):
            off = base + j * CH
            pltpu.sync_copy(gidx_hbm.at[pl.ds(off, CH)], g)
            pltpu.sync_copy(sidx_hbm.at[pl.ds(off, CH)], s)
            pltpu.async_copy(x_hbm.at[g], r, m)

        def drain_add(g, s, r, m):
            pltpu.make_async_copy(x_hbm.at[g], r, m).wait()
            pltpu.sync_copy(r, acc.at[s], add=True)

        fetch(0, g0, s0, r0, m0)

        def pair(p, carry):
            j = 2 * p
            fetch(j + 1, g1, s1, r1, m1)
            drain_add(g0, s0, r0, m0)
            fetch(j + 2, g0, s0, r0, m0)
            drain_add(g1, s1, r1, m1)
            return carry
        lax.fori_loop(0, (T - 1) // 2, pair, 0)
        drain_add(g0, s0, r0, m0)

        plsc.subcore_barrier()
        pltpu.sync_copy(acc.at[pl.ds(row0, RP)],
                        gsum_hbm.at[pl.ds(cid * NP + row0, RP)])

    return pl.kernel(body, out_type=out_type, mesh=mesh,
                     scratch_types=scratch)


@functools.lru_cache(maxsize=None)
def _sc_scatter_fn():
    return _make_sc_scatter()


def kernel(node_features, edge_index, Ws0, bs0, Wc0, bc0, A1_0, b1_0, A2_0,
           g0, be0, rm0, rv0, Ws1, bs1, Wc1, bc1, A1_1, b1_1, A2_1, g1, be1,
           rm1, rv1, D1, d1b, D2, d2b):
    x = node_features
    src = edge_index[0]
    dst = edge_index[1]
    gidx = jnp.concatenate([src, dst])
    sidx = jnp.concatenate([dst, src])

    (g0s,) = _sc_scatter_fn()(x, gidx, sidx)
    g0s = g0s.reshape(2, NP, H)
    gin0, gout0 = g0s[0, :N], g0s[1, :N]
    ones_e = jnp.ones((E,), jnp.float32)
    din = jnp.zeros((N,), jnp.float32).at[dst].add(ones_e)
    dout = jnp.zeros((N,), jnp.float32).at[src].add(ones_e)
    din16 = jnp.broadcast_to(din[:, None], (N, 16))
    dout16 = jnp.broadcast_to(dout[:, None], (N, 16))
    x1, att = _dense_layer0(x, gin0, gout0, din16, dout16, Ws0, bs0, Wc0,
                            bc0, A1_0, b1_0, A2_0, g0, be0, rm0, rv0)
    (g1s,) = _sc_scatter_fn()(x1, gidx, sidx)
    g1s = g1s.reshape(2, NP, H)
    logp = _dense_layer1(x1, g1s[0, :N], g1s[1, :N], din16, dout16, Ws1, bs1,
                         Wc1, bc1, A1_1, b1_1, A2_1, g1, be1, rm1, rv1,
                         D1, d1b, D2, d2b)
    return (logp, att.reshape(N, 3, 1))
